# baseline (speedup 1.0000x reference)
import functools

import jax
import jax.numpy as jnp
from jax import lax
from jax.experimental import pallas as pl
from jax.experimental.pallas import tpu as pltpu
from jax.experimental.pallas import tpu_sc as plsc

EMB = 64
SEQ = 200
NC, NS, L = 2, 16, 16
NW = NC * NS
BB = 128
NBUF = 5
EG = EMB // 8
PITCH = BB + 1


def _emb_kernel(batch):
    n_blocks = batch // BB
    assert n_blocks == NW
    mesh = plsc.VectorSubcoreMesh(
        core_axis_name="c", subcore_axis_name="s", num_cores=NC, num_subcores=NS
    )

    @functools.partial(
        pl.kernel,
        out_type=jax.ShapeDtypeStruct((SEQ, EG, NW, 8, BB), jnp.float32),
        mesh=mesh,
        scratch_types=[
            pltpu.VMEM((SEQ, BB), jnp.int32),
            pltpu.VMEM((NBUF, BB, EMB), jnp.float32),
            pltpu.VMEM((NBUF, EG, 8, PITCH), jnp.float32),
            pltpu.VMEM((SEQ, EMB), jnp.float32),
            pltpu.SemaphoreType.DMA((NBUF,)),
            pltpu.SemaphoreType.DMA((NBUF,)),
        ],
        compiler_params=pltpu.CompilerParams(
            use_tc_tiling_on_sc=False, needs_layout_passes=False),
    )
    def body(tokt_hbm, tab_hbm, pos_hbm, out_hbm, idx_all, gbuf, wbuf, posv,
             gsem, wsem):
        wid = lax.axis_index("s") * NC + lax.axis_index("c")
        col = pl.multiple_of(wid * BB, 8)
        pltpu.sync_copy(tokt_hbm.at[:, pl.ds(col, BB)], idx_all)
        pltpu.sync_copy(pos_hbm, posv)

        evecs = [jnp.arange(L, dtype=jnp.int32) + L * g for g in range(EMB // L)]
        egvecs = [e // 8 for e in evecs]
        e8vecs = [e % 8 for e in evecs]

        def gather(s, b):
            pltpu.async_copy(tab_hbm.at[idx_all.at[s]], gbuf.at[b], gsem.at[b])

        def gather_wait(s, b):
            pltpu.make_async_copy(tab_hbm.at[idx_all.at[s]], gbuf.at[b],
                                  gsem.at[b]).wait()

        def write(s, b):
            pltpu.async_copy(
                wbuf.at[b, :, :, pl.ds(0, BB)],
                out_hbm.at[s, :, wid], wsem.at[b])

        def write_wait(s, b):
            pltpu.make_async_copy(
                wbuf.at[b, :, :, pl.ds(0, BB)],
                out_hbm.at[s, :, wid], wsem.at[b]).wait()

        for b in range(NBUF - 1):
            gather(b, b)

        def group_body(q, carry):
            for b in range(NBUF):
                s = q * NBUF + b
                bp = (b + NBUF - 1) % NBUF

                @pl.when(s + NBUF - 1 < SEQ)
                def _():
                    @pl.when(s >= 1)
                    def _():
                        write_wait(s - 1, bp)

                    gather(s + NBUF - 1, bp)

                gather_wait(s, b)

                @pl.loop(0, BB, unroll=4)
                def row_body(r):
                    rvec = jnp.full((L,), 0, jnp.int32) + r
                    for g in range(EMB // L):
                        sl = pl.ds(g * L, L)
                        val = gbuf[b, r, sl] + posv[s, sl]
                        plsc.store_scatter(
                            wbuf.at[b], [egvecs[g], e8vecs[g], rvec], val)

                write(s, b)
            return carry

        lax.fori_loop(0, SEQ // NBUF, group_body, 0)
        for b in range(NBUF):
            write_wait(SEQ - NBUF + b, (SEQ - NBUF + b) % NBUF)

    return body


def kernel(tokens, static_table, pos_table):
    batch, seq = tokens.shape
    tokt = tokens.T.astype(jnp.int32)
    out5d = _emb_kernel(batch)(tokt, static_table, pos_table)
    out = out5d.transpose(2, 4, 0, 1, 3).reshape(batch, seq, EMB)
    return out


# speedup vs baseline: 1.0373x; 1.0373x over previous
import functools

import jax
import jax.numpy as jnp
from jax import lax
from jax.experimental import pallas as pl
from jax.experimental.pallas import tpu as pltpu
from jax.experimental.pallas import tpu_sc as plsc

EMB = 64
SEQ = 200
NC, NS, L = 2, 16, 16
NW = NC * NS
BB = 128
NBUF = 5
EG = EMB // 8
PITCH = BB + 1


def _emb_kernel(batch):
    n_blocks = batch // BB
    assert n_blocks == NW
    mesh = plsc.VectorSubcoreMesh(
        core_axis_name="c", subcore_axis_name="s", num_cores=NC, num_subcores=NS
    )

    @functools.partial(
        pl.kernel,
        out_type=jax.ShapeDtypeStruct((SEQ, EG, NW, 8, BB), jnp.float32),
        mesh=mesh,
        scratch_types=[
            pltpu.VMEM((SEQ, BB), jnp.int32),
            pltpu.VMEM((NBUF, BB, EMB), jnp.float32),
            pltpu.VMEM((NBUF, EMB, PITCH), jnp.float32),
            pltpu.VMEM((SEQ, EMB), jnp.float32),
            pltpu.SemaphoreType.DMA((NBUF,)),
            pltpu.SemaphoreType.DMA((NBUF,)),
        ],
        compiler_params=pltpu.CompilerParams(
            use_tc_tiling_on_sc=False, needs_layout_passes=False),
    )
    def body(tokt_hbm, tab_hbm, pos_hbm, out_hbm, idx_all, gbuf, wbuf, posv,
             gsem, wsem):
        wid = lax.axis_index("s") * NC + lax.axis_index("c")
        col = pl.multiple_of(wid * BB, 8)
        pltpu.sync_copy(tokt_hbm.at[:, pl.ds(col, BB)], idx_all)
        pltpu.sync_copy(pos_hbm, posv)

        evecs = [jnp.arange(L, dtype=jnp.int32) + L * g for g in range(EMB // L)]

        def gather(s, b):
            pltpu.async_copy(tab_hbm.at[idx_all.at[s]], gbuf.at[b], gsem.at[b])

        def gather_wait(s, b):
            pltpu.make_async_copy(tab_hbm.at[idx_all.at[s]], gbuf.at[b],
                                  gsem.at[b]).wait()

        def write(s, b):
            for eg in range(EG):
                pltpu.async_copy(
                    wbuf.at[b, pl.ds(eg * 8, 8), pl.ds(0, BB)],
                    out_hbm.at[s, eg, wid], wsem.at[b])

        def write_wait(s, b):
            for eg in range(EG):
                pltpu.make_async_copy(
                    wbuf.at[b, pl.ds(eg * 8, 8), pl.ds(0, BB)],
                    out_hbm.at[s, eg, wid], wsem.at[b]).wait()

        for b in range(NBUF - 1):
            gather(b, b)

        def group_body(q, carry):
            for b in range(NBUF):
                s = q * NBUF + b
                bp = (b + NBUF - 1) % NBUF

                @pl.when(s + NBUF - 1 < SEQ)
                def _():
                    @pl.when(s >= 1)
                    def _():
                        write_wait(s - 1, bp)

                    gather(s + NBUF - 1, bp)

                gather_wait(s, b)
                pvecs = [posv[s, pl.ds(g * L, L)] for g in range(EMB // L)]

                @pl.loop(0, BB, unroll=8)
                def row_body(r):
                    rvec = jnp.full((L,), 0, jnp.int32) + r
                    for g in range(EMB // L):
                        val = gbuf[b, r, pl.ds(g * L, L)] + pvecs[g]
                        plsc.store_scatter(
                            wbuf.at[b], [evecs[g], rvec], val)

                write(s, b)
            return carry

        lax.fori_loop(0, SEQ // NBUF, group_body, 0)
        for b in range(NBUF):
            write_wait(SEQ - NBUF + b, (SEQ - NBUF + b) % NBUF)

    return body


def kernel(tokens, static_table, pos_table):
    batch, seq = tokens.shape
    tokt = tokens.T.astype(jnp.int32)
    out5d = _emb_kernel(batch)(tokt, static_table, pos_table)
    out = out5d.transpose(2, 4, 0, 1, 3).reshape(batch, seq, EMB)
    return out
